# baseline (speedup 1.0000x reference)
import functools

import jax
import jax.numpy as jnp
from jax import lax
from jax.experimental import pallas as pl
from jax.experimental.pallas import tpu as pltpu
from jax.experimental.pallas import tpu_sc as plsc

E = 16384
L = 50
D = 64
NC = 2
NS = 16
NW = NC * NS
EPW = E // NW
HALF = EPW // 2
LANE = 16
NG = EPW // LANE
NB = 2


def _sc_body(tok_hbm, c1_hbm, c2_hbm, c3_hbm,
             tok_v, c1_v, c2_v, c3_v, osem):
    wid = lax.axis_index("s") * NC + lax.axis_index("c")
    base_e = wid * EPW
    pltpu.sync_copy(tok_hbm.at[:, pl.ds(base_e, EPW)], tok_v)

    @plsc.parallel_loop(0, NG // NB)
    def block_body(bi):
        base = bi * (NB * LANE)
        zero = jnp.zeros((LANE,), jnp.int32)

        def l_body(l, acc):
            sb0, sb1, c3 = acc
            nsb0, nsb1, nc3 = [], [], []
            for k in range(NB):
                t = tok_v[l, pl.ds(base + k * LANE, LANE)]
                b0 = t & 1
                b1 = t >> 1
                nsb0.append(sb0[k] + b0)
                nsb1.append(sb1[k] + b1)
                nc3.append(c3[k] + (b0 & b1))
            return tuple(nsb0), tuple(nsb1), tuple(nc3)

        sb0, sb1, c3 = lax.fori_loop(
            0, L, l_body, ((zero,) * NB, (zero,) * NB, (zero,) * NB))
        for k in range(NB):
            gs = base + k * LANE
            c1_v[pl.ds(gs, LANE)] = (sb0[k] - c3[k]).astype(jnp.float32)
            c2_v[pl.ds(gs, LANE)] = (sb1[k] - c3[k]).astype(jnp.float32)
            c3_v[pl.ds(gs, LANE)] = c3[k].astype(jnp.float32)

    o1 = pltpu.async_copy(c1_v, c1_hbm.at[pl.ds(base_e, EPW)], osem)
    o2 = pltpu.async_copy(c2_v, c2_hbm.at[pl.ds(base_e, EPW)], osem)
    o3 = pltpu.async_copy(c3_v, c3_hbm.at[pl.ds(base_e, EPW)], osem)
    o1.wait()
    o2.wait()
    o3.wait()


_sc_counts = functools.partial(
    pl.kernel,
    out_type=[jax.ShapeDtypeStruct((E,), jnp.float32)] * 3,
    mesh=plsc.VectorSubcoreMesh(
        core_axis_name="c", subcore_axis_name="s",
        num_cores=NC, num_subcores=NS),
    compiler_params=pltpu.CompilerParams(needs_layout_passes=False),
    scratch_types=[
        pltpu.VMEM((L, EPW), jnp.int32),
        pltpu.VMEM((EPW,), jnp.float32),
        pltpu.VMEM((EPW,), jnp.float32),
        pltpu.VMEM((EPW,), jnp.float32),
        pltpu.SemaphoreType.DMA,
    ],
)(_sc_body)


BE = 8192


def _tc_body(c1_ref, c2_ref, c3_ref, emb_ref, outt_ref):
    embt = emb_ref[...].T
    pb = embt[:, 0:1]
    inv_l = 1.0 / L
    d1 = (embt[:, 1:2] - pb) * inv_l
    d2 = (embt[:, 2:3] - pb) * inv_l
    d3 = (embt[:, 3:4] - pb) * inv_l
    c1 = c1_ref[...][None, :]
    c2 = c2_ref[...][None, :]
    c3 = c3_ref[...][None, :]
    outt_ref[...] = pb + d1 * c1 + d2 * c2 + d3 * c3


def _tc_expand(c1, c2, c3, emb_table):
    return pl.pallas_call(
        _tc_body,
        grid=(E // BE,),
        in_specs=[
            pl.BlockSpec((BE,), lambda i: (i,)),
            pl.BlockSpec((BE,), lambda i: (i,)),
            pl.BlockSpec((BE,), lambda i: (i,)),
            pl.BlockSpec((4, D), lambda i: (0, 0)),
        ],
        out_specs=pl.BlockSpec((D, BE), lambda i: (0, i)),
        out_shape=jax.ShapeDtypeStruct((D, E), jnp.float32),
    )(c1, c2, c3, emb_table)


def kernel(overlap_similarity, overlap_length, reads_tokens, emb_table, W, b):
    c1, c2, c3 = _sc_counts(reads_tokens.T)
    return _tc_expand(c1, c2, c3, emb_table).T


# speedup vs baseline: 1.0332x; 1.0332x over previous
import functools

import jax
import jax.numpy as jnp
from jax import lax
from jax.experimental import pallas as pl
from jax.experimental.pallas import tpu as pltpu
from jax.experimental.pallas import tpu_sc as plsc

E = 16384
L = 50
D = 64
NC = 2
NS = 16
NW = NC * NS
EPW = E // NW
LANE = 16
NG = EPW // LANE
NB = 4


def _sc_body(tok_hbm, c1_hbm, c2_hbm, c3_hbm,
             tok_v, c1_v, c2_v, c3_v, osem):
    wid = lax.axis_index("s") * NC + lax.axis_index("c")
    base_e = wid * EPW
    pltpu.sync_copy(tok_hbm.at[:, pl.ds(base_e, EPW)], tok_v)

    @plsc.parallel_loop(0, NG // NB)
    def block_body(bi):
        base = bi * (NB * LANE)
        zero = jnp.zeros((LANE,), jnp.int32)

        def l_body(l, acc):
            sb0, sb1, c3 = acc
            nsb0, nsb1, nc3 = [], [], []
            for k in range(NB):
                t = tok_v[l, pl.ds(base + k * LANE, LANE)]
                b0 = t & 1
                b1 = t >> 1
                nsb0.append(sb0[k] + b0)
                nsb1.append(sb1[k] + b1)
                nc3.append(c3[k] + (b0 & b1))
            return tuple(nsb0), tuple(nsb1), tuple(nc3)

        sb0, sb1, c3 = lax.fori_loop(
            0, L, l_body, ((zero,) * NB, (zero,) * NB, (zero,) * NB))
        for k in range(NB):
            gs = base + k * LANE
            c1_v[pl.ds(gs, LANE)] = (sb0[k] - c3[k]).astype(jnp.float32)
            c2_v[pl.ds(gs, LANE)] = (sb1[k] - c3[k]).astype(jnp.float32)
            c3_v[pl.ds(gs, LANE)] = c3[k].astype(jnp.float32)

    o1 = pltpu.async_copy(c1_v, c1_hbm.at[pl.ds(base_e, EPW)], osem)
    o2 = pltpu.async_copy(c2_v, c2_hbm.at[pl.ds(base_e, EPW)], osem)
    o3 = pltpu.async_copy(c3_v, c3_hbm.at[pl.ds(base_e, EPW)], osem)
    o1.wait()
    o2.wait()
    o3.wait()


_sc_counts = functools.partial(
    pl.kernel,
    out_type=[jax.ShapeDtypeStruct((E,), jnp.float32)] * 3,
    mesh=plsc.VectorSubcoreMesh(
        core_axis_name="c", subcore_axis_name="s",
        num_cores=NC, num_subcores=NS),
    compiler_params=pltpu.CompilerParams(needs_layout_passes=False),
    scratch_types=[
        pltpu.VMEM((L, EPW), jnp.int32),
        pltpu.VMEM((EPW,), jnp.float32),
        pltpu.VMEM((EPW,), jnp.float32),
        pltpu.VMEM((EPW,), jnp.float32),
        pltpu.SemaphoreType.DMA,
    ],
)(_sc_body)


BE = 8192


def _tc_body(c1_ref, c2_ref, c3_ref, emb_ref, outt_ref):
    embt = emb_ref[...].T
    pb = embt[:, 0:1]
    inv_l = 1.0 / L
    d1 = (embt[:, 1:2] - pb) * inv_l
    d2 = (embt[:, 2:3] - pb) * inv_l
    d3 = (embt[:, 3:4] - pb) * inv_l
    c1 = c1_ref[...][None, :]
    c2 = c2_ref[...][None, :]
    c3 = c3_ref[...][None, :]
    outt_ref[...] = pb + d1 * c1 + d2 * c2 + d3 * c3


def _tc_expand(c1, c2, c3, emb_table):
    return pl.pallas_call(
        _tc_body,
        grid=(E // BE,),
        in_specs=[
            pl.BlockSpec((BE,), lambda i: (i,)),
            pl.BlockSpec((BE,), lambda i: (i,)),
            pl.BlockSpec((BE,), lambda i: (i,)),
            pl.BlockSpec((4, D), lambda i: (0, 0)),
        ],
        out_specs=pl.BlockSpec((D, BE), lambda i: (0, i)),
        out_shape=jax.ShapeDtypeStruct((D, E), jnp.float32),
    )(c1, c2, c3, emb_table)


def kernel(overlap_similarity, overlap_length, reads_tokens, emb_table, W, b):
    c1, c2, c3 = _sc_counts(reads_tokens.T)
    return _tc_expand(c1, c2, c3, emb_table).T
